# baseline (speedup 1.0000x reference)
import functools

import jax
import jax.numpy as jnp
from jax import lax
from jax.experimental import pallas as pl
from jax.experimental.pallas import tpu as pltpu
from jax.experimental.pallas import tpu_sc as plsc

NC = 2
NS = 16
CH = 125


def _make_sc_agg(n_nodes, n_edges, feat2, with_deg, final=False):
    e_w = n_edges // NS
    n_ch = e_w // CH
    rpt = (n_nodes // NS) & ~7
    rem = n_nodes - rpt * NS
    FB = rpt // 4

    out_type = [jax.ShapeDtypeStruct((NC, n_nodes, feat2), jnp.float32)]
    scratch = [
        pltpu.VMEM((n_ch, CH), jnp.int32),
        pltpu.VMEM((n_ch, CH), jnp.int32),
    ] + [pltpu.VMEM((CH, feat2), jnp.float32) for _ in range(4)] + [
        pltpu.VMEM_SHARED((n_nodes, feat2), jnp.float32),
    ] + [pltpu.SemaphoreType.DMA for _ in range(8)]
    if with_deg:
        out_type.append(jax.ShapeDtypeStruct((NC, n_nodes, 16), jnp.float32))
        scratch += [
            pltpu.VMEM((CH, 16), jnp.float32),
            pltpu.VMEM_SHARED((n_nodes, 16), jnp.float32),
        ]
    if final:
        scratch += [
            pltpu.VMEM((FB, feat2), jnp.float32),
            pltpu.VMEM((FB, 16), jnp.float32),
            pltpu.VMEM((FB, feat2), jnp.float32),
        ]

    mesh = plsc.VectorSubcoreMesh(core_axis_name="c", subcore_axis_name="s")

    half = n_ch // NC

    def body(g_hbm, src_hbm, dst_hbm, zf_hbm, *rest):
        sk_hbm = rt_hbm = sbuf = rbuf = abuf = None
        if with_deg:
            (ones_hbm, z16_hbm, out_hbm, deg_hbm, si, di,
             r0, r1, r2, r3, acc, g0, g1, g2, g3, s0, s1, s2, s3,
             ones_v, dacc) = rest
        elif final:
            (sk_hbm, rt_hbm, out_hbm, si, di, r0, r1, r2, r3, acc,
             g0, g1, g2, g3, s0, s1, s2, s3, sbuf, rbuf, abuf) = rest
        else:
            (out_hbm, si, di, r0, r1, r2, r3, acc,
             g0, g1, g2, g3, s0, s1, s2, s3) = rest
        rows = (r0, r1, r2, r3)
        gsem = (g0, g1, g2, g3)
        ssem = (s0, s1, s2, s3)
        c = lax.axis_index("c")
        s = lax.axis_index("s")
        pltpu.sync_copy(src_hbm.at[s], si)
        pltpu.sync_copy(dst_hbm.at[s], di)
        pltpu.sync_copy(zf_hbm.at[pl.ds(0, rpt)], acc.at[pl.ds(s * rpt, rpt)])
        if with_deg:
            pltpu.sync_copy(ones_hbm, ones_v)
            pltpu.sync_copy(z16_hbm.at[pl.ds(0, rpt)],
                            dacc.at[pl.ds(s * rpt, rpt)])

        @pl.when(s == NS - 1)
        def _zero_tail():
            pltpu.sync_copy(zf_hbm.at[pl.ds(0, rem)],
                            acc.at[pl.ds(NS * rpt, rem)])
            if with_deg:
                pltpu.sync_copy(z16_hbm.at[pl.ds(0, rem)],
                                dacc.at[pl.ds(NS * rpt, rem)])

        plsc.subcore_barrier()

        def wait_gather(j, b):
            pltpu.make_async_copy(
                g_hbm.at[c].at[si.at[j]], rows[b], gsem[b]).wait()

        def wait_scatter(j, b):
            pltpu.make_async_copy(rows[b], acc.at[di.at[j]], ssem[b]).wait()

        for b in range(2):
            pltpu.async_copy(g_hbm.at[c].at[si.at[b]], rows[b], gsem[b])

        def step(t, carry):
            for b4 in range(4):
                j = 4 * t + b4
                b = b4 % 4
                wait_gather(j, b)
                pltpu.async_copy(rows[b], acc.at[di.at[j]], ssem[b], add=True)
                if with_deg:
                    @pl.when((j >= c * half) & (j < (c + 1) * half))
                    def _deg_add():
                        pltpu.sync_copy(ones_v, dacc.at[di.at[j]], add=True)

                nb = (b + 2) % 4

                @pl.when(j + 2 < n_ch)
                def _next_gather():
                    @pl.when(j >= 2)
                    def _drain_prev():
                        wait_scatter(j - 2, nb)

                    pltpu.async_copy(
                        g_hbm.at[c].at[si.at[j + 2]], rows[nb], gsem[nb])
            return carry

        lax.fori_loop(0, n_ch // 4, step, 0)
        for k in range(4):
            j = n_ch - 4 + k
            wait_scatter(j, j % 4)
        plsc.subcore_barrier()

        if final:
            def combine(base, nrows):
                bs = pl.ds(0, nrows)
                pltpu.sync_copy(sk_hbm.at[c, pl.ds(base, nrows)], sbuf.at[bs])
                pltpu.sync_copy(rt_hbm.at[pl.ds(base, nrows)], rbuf.at[bs])
                pltpu.sync_copy(acc.at[pl.ds(base, nrows)], abuf.at[bs])

                def rowfn(r, carry):
                    rv = rbuf[r, pl.ds(0, 16)]
                    for g in range(feat2 // 16):
                        cs = pl.ds(g * 16, 16)
                        sbuf[r, cs] = sbuf[r, cs] + abuf[r, cs] * rv
                    return carry

                lax.fori_loop(0, nrows, rowfn, 0)
                pltpu.sync_copy(sbuf.at[bs], out_hbm.at[c, pl.ds(base, nrows)])

            for blk in range(4):
                combine(s * rpt + blk * FB, FB)

            @pl.when(s == NS - 1)
            def _fin_tail():
                combine(NS * rpt, rem)
        else:
            sl = pl.ds(s * rpt, rpt)
            pltpu.sync_copy(acc.at[sl], out_hbm.at[c, sl])
            if with_deg:
                pltpu.sync_copy(dacc.at[sl], deg_hbm.at[c, sl])

            @pl.when(s == NS - 1)
            def _write_tail():
                tl = pl.ds(NS * rpt, rem)
                pltpu.sync_copy(acc.at[tl], out_hbm.at[c, tl])
                if with_deg:
                    pltpu.sync_copy(dacc.at[tl], deg_hbm.at[c, tl])

    return pl.kernel(
        body, out_type=out_type, mesh=mesh, scratch_types=scratch,
        compiler_params=pltpu.CompilerParams(use_tc_tiling_on_sc=False))


def _mm_body(x_ref, w_ref, o_ref):
    v = jnp.dot(x_ref[...], w_ref[...], preferred_element_type=jnp.float32)
    h = v.shape[1] // 2
    o_ref[0] = v[:, :h]
    o_ref[1] = v[:, h:]


def _mid_body(x_ref, p1_ref, deg_ref, ws1, b1r, wn2, ws2, wcp, b2r, bcr,
              p_out, s_out, rt_out):
    deg16 = jnp.maximum(deg_ref[0] + deg_ref[1], 1.0)
    rt_out[...] = 1.0 / deg16
    r = rt_out[:, :1]
    agg = jnp.concatenate([p1_ref[0], p1_ref[1]], axis=1) * r
    h = x_ref[...] @ ws1[...] + agg + b1r[...]
    h = jnp.maximum(h, 0.0)
    dot = functools.partial(jnp.dot, preferred_element_type=jnp.float32)
    p = dot(dot(h, wn2[...]), wcp[...])
    sv = dot(dot(h, ws2[...]), wcp[...]) + dot(b2r[...], wcp[...]) + bcr[...]
    hp = p.shape[1] // 2
    p_out[0] = p[:, :hp]
    p_out[1] = p[:, hp:]
    s_out[0] = sv[:, :hp]
    s_out[1] = sv[:, hp:]


def kernel(x, edge_index, W_self1, W_nei1, b1, W_self2, W_nei2, b2, Wc, bc):
    n, f = x.shape
    n_edges = edge_index.shape[1]
    ncls = Wc.shape[1]
    fp = 64
    f2 = f // 2
    fp2 = fp // 2
    e_w = n_edges // NS
    n_ch = e_w // CH
    rpt_buf = -(-n // NS)

    src = edge_index[0].astype(jnp.int32).reshape(NS, n_ch, CH)
    dst = edge_index[1].astype(jnp.int32).reshape(NS, n_ch, CH)
    zf = jnp.zeros((rpt_buf, f2), jnp.float32)
    z16 = jnp.zeros((rpt_buf, 16), jnp.float32)
    zfp = jnp.zeros((rpt_buf, fp2), jnp.float32)
    ones = jnp.ones((CH, 16), jnp.float32)
    wcp = jnp.concatenate([Wc, jnp.zeros((f, fp - ncls), jnp.float32)], axis=1)
    bcp = jnp.concatenate([bc, jnp.zeros((fp - ncls,), jnp.float32)]).reshape(1, fp)
    b1r = b1.reshape(1, f)
    b2r = b2.reshape(1, f)

    BLK = 1000
    grid = (n // BLK,)

    g1 = pl.pallas_call(
        _mm_body,
        grid=grid,
        in_specs=[pl.BlockSpec((BLK, f), lambda i: (i, 0)),
                  pl.BlockSpec((f, f), lambda i: (0, 0))],
        out_specs=pl.BlockSpec((2, BLK, f2), lambda i: (0, i, 0)),
        out_shape=jax.ShapeDtypeStruct((2, n, f2), jnp.float32),
    )(x, W_nei1)

    agg1, deg = _make_sc_agg(n, n_edges, f2, True)(g1, src, dst, zf, ones, z16)

    p, s, rt = pl.pallas_call(
        _mid_body,
        grid=grid,
        in_specs=[
            pl.BlockSpec((BLK, f), lambda i: (i, 0)),
            pl.BlockSpec((2, BLK, f2), lambda i: (0, i, 0)),
            pl.BlockSpec((2, BLK, 16), lambda i: (0, i, 0)),
            pl.BlockSpec((f, f), lambda i: (0, 0)),
            pl.BlockSpec((1, f), lambda i: (0, 0)),
            pl.BlockSpec((f, f), lambda i: (0, 0)),
            pl.BlockSpec((f, f), lambda i: (0, 0)),
            pl.BlockSpec((f, fp), lambda i: (0, 0)),
            pl.BlockSpec((1, f), lambda i: (0, 0)),
            pl.BlockSpec((1, fp), lambda i: (0, 0)),
        ],
        out_specs=[pl.BlockSpec((2, BLK, fp2), lambda i: (0, i, 0)),
                   pl.BlockSpec((2, BLK, fp2), lambda i: (0, i, 0)),
                   pl.BlockSpec((BLK, 16), lambda i: (i, 0))],
        out_shape=[jax.ShapeDtypeStruct((2, n, fp2), jnp.float32),
                   jax.ShapeDtypeStruct((2, n, fp2), jnp.float32),
                   jax.ShapeDtypeStruct((n, 16), jnp.float32)],
    )(x, agg1, deg, W_self1, b1r, W_nei2, W_self2, wcp, b2r, bcp)

    (out2,) = _make_sc_agg(n, n_edges, fp2, False, final=True)(
        p, src, dst, zfp, s, rt)

    return jnp.concatenate([out2[0], out2[1]], axis=1)[:, :ncls]


# speedup vs baseline: 1.0043x; 1.0043x over previous
import functools

import jax
import jax.numpy as jnp
from jax import lax
from jax.experimental import pallas as pl
from jax.experimental.pallas import tpu as pltpu
from jax.experimental.pallas import tpu_sc as plsc

NC = 2
NS = 16
CH = 125


def _make_sc_agg(n_nodes, n_edges, feat2, with_deg, final=False,
                 packed_cols=False):
    e_w = n_edges // NS
    n_ch = e_w // CH
    rpt = (n_nodes // NS) & ~7
    rem = n_nodes - rpt * NS
    FB = rpt // 4

    out_type = [jax.ShapeDtypeStruct((NC, n_nodes, feat2), jnp.float32)]
    scratch = [
        pltpu.VMEM((n_ch, CH), jnp.int32),
        pltpu.VMEM((n_ch, CH), jnp.int32),
    ] + [pltpu.VMEM((CH, feat2), jnp.float32) for _ in range(4)] + [
        pltpu.VMEM_SHARED((n_nodes, feat2), jnp.float32),
    ] + [pltpu.SemaphoreType.DMA for _ in range(8)]
    if with_deg:
        out_type.append(jax.ShapeDtypeStruct((NC, n_nodes, 16), jnp.float32))
        scratch += [
            pltpu.VMEM((CH, 16), jnp.float32),
            pltpu.VMEM_SHARED((n_nodes, 16), jnp.float32),
        ]
    if final:
        scratch += [
            pltpu.VMEM((FB, feat2), jnp.float32),
            pltpu.VMEM((FB, 16), jnp.float32),
            pltpu.VMEM((FB, feat2), jnp.float32),
        ]

    mesh = plsc.VectorSubcoreMesh(core_axis_name="c", subcore_axis_name="s")

    half = n_ch // NC

    def body(g_hbm, src_hbm, dst_hbm, zf_hbm, *rest):
        sk_hbm = rt_hbm = sbuf = rbuf = abuf = None
        if with_deg:
            (ones_hbm, z16_hbm, out_hbm, deg_hbm, si, di,
             r0, r1, r2, r3, acc, g0, g1, g2, g3, s0, s1, s2, s3,
             ones_v, dacc) = rest
        elif final:
            (sk_hbm, rt_hbm, out_hbm, si, di, r0, r1, r2, r3, acc,
             g0, g1, g2, g3, s0, s1, s2, s3, sbuf, rbuf, abuf) = rest
        else:
            (out_hbm, si, di, r0, r1, r2, r3, acc,
             g0, g1, g2, g3, s0, s1, s2, s3) = rest
        rows = (r0, r1, r2, r3)
        gsem = (g0, g1, g2, g3)
        ssem = (s0, s1, s2, s3)
        c = lax.axis_index("c")
        s = lax.axis_index("s")
        pltpu.sync_copy(src_hbm.at[s], si)
        pltpu.sync_copy(dst_hbm.at[s], di)
        pltpu.sync_copy(zf_hbm.at[pl.ds(0, rpt)], acc.at[pl.ds(s * rpt, rpt)])
        if with_deg:
            pltpu.sync_copy(ones_hbm, ones_v)
            pltpu.sync_copy(z16_hbm.at[pl.ds(0, rpt)],
                            dacc.at[pl.ds(s * rpt, rpt)])

        @pl.when(s == NS - 1)
        def _zero_tail():
            pltpu.sync_copy(zf_hbm.at[pl.ds(0, rem)],
                            acc.at[pl.ds(NS * rpt, rem)])
            if with_deg:
                pltpu.sync_copy(z16_hbm.at[pl.ds(0, rem)],
                                dacc.at[pl.ds(NS * rpt, rem)])

        plsc.subcore_barrier()

        if packed_cols:
            g_half = g_hbm.at[:, pl.ds(c * feat2, feat2)]
        else:
            g_half = g_hbm.at[c]

        def wait_gather(j, b):
            pltpu.make_async_copy(
                g_half.at[si.at[j]], rows[b], gsem[b]).wait()

        def wait_scatter(j, b):
            pltpu.make_async_copy(rows[b], acc.at[di.at[j]], ssem[b]).wait()

        for b in range(2):
            pltpu.async_copy(g_half.at[si.at[b]], rows[b], gsem[b])

        def step(t, carry):
            for b4 in range(4):
                j = 4 * t + b4
                b = b4 % 4
                wait_gather(j, b)
                pltpu.async_copy(rows[b], acc.at[di.at[j]], ssem[b], add=True)
                if with_deg:
                    @pl.when((j >= c * half) & (j < (c + 1) * half))
                    def _deg_add():
                        pltpu.sync_copy(ones_v, dacc.at[di.at[j]], add=True)

                nb = (b + 2) % 4

                @pl.when(j + 2 < n_ch)
                def _next_gather():
                    @pl.when(j >= 2)
                    def _drain_prev():
                        wait_scatter(j - 2, nb)

                    pltpu.async_copy(
                        g_half.at[si.at[j + 2]], rows[nb], gsem[nb])
            return carry

        lax.fori_loop(0, n_ch // 4, step, 0)
        for k in range(4):
            j = n_ch - 4 + k
            wait_scatter(j, j % 4)
        plsc.subcore_barrier()

        if final:
            def combine(base, nrows):
                bs = pl.ds(0, nrows)
                pltpu.sync_copy(sk_hbm.at[c, pl.ds(base, nrows)], sbuf.at[bs])
                pltpu.sync_copy(rt_hbm.at[pl.ds(base, nrows)], rbuf.at[bs])
                pltpu.sync_copy(acc.at[pl.ds(base, nrows)], abuf.at[bs])

                def rowfn(r, carry):
                    rv = rbuf[r, pl.ds(0, 16)]
                    for g in range(feat2 // 16):
                        cs = pl.ds(g * 16, 16)
                        sbuf[r, cs] = sbuf[r, cs] + abuf[r, cs] * rv
                    return carry

                lax.fori_loop(0, nrows, rowfn, 0)
                pltpu.sync_copy(sbuf.at[bs], out_hbm.at[c, pl.ds(base, nrows)])

            for blk in range(4):
                combine(s * rpt + blk * FB, FB)

            @pl.when(s == NS - 1)
            def _fin_tail():
                combine(NS * rpt, rem)
        else:
            sl = pl.ds(s * rpt, rpt)
            pltpu.sync_copy(acc.at[sl], out_hbm.at[c, sl])
            if with_deg:
                pltpu.sync_copy(dacc.at[sl], deg_hbm.at[c, sl])

            @pl.when(s == NS - 1)
            def _write_tail():
                tl = pl.ds(NS * rpt, rem)
                pltpu.sync_copy(acc.at[tl], out_hbm.at[c, tl])
                if with_deg:
                    pltpu.sync_copy(dacc.at[tl], deg_hbm.at[c, tl])

    return pl.kernel(
        body, out_type=out_type, mesh=mesh, scratch_types=scratch,
        compiler_params=pltpu.CompilerParams(use_tc_tiling_on_sc=False))


def _mid_body(x_ref, ax_ref, deg_ref, ws1, b1r, wn1, wn2, ws2, wcp, b2r, bcr,
              p_out, s_out, rt_out):
    deg16 = jnp.maximum(deg_ref[0] + deg_ref[1], 1.0)
    rt_out[...] = 1.0 / deg16
    r = rt_out[:, :1]
    aggx = jnp.concatenate([ax_ref[0], ax_ref[1]], axis=1) * r
    dot = functools.partial(jnp.dot, preferred_element_type=jnp.float32)
    h = x_ref[...] @ ws1[...] + dot(aggx, wn1[...]) + b1r[...]
    h = jnp.maximum(h, 0.0)
    p = dot(dot(h, wn2[...]), wcp[...])
    sv = dot(dot(h, ws2[...]), wcp[...]) + dot(b2r[...], wcp[...]) + bcr[...]
    hp = p.shape[1] // 2
    p_out[0] = p[:, :hp]
    p_out[1] = p[:, hp:]
    s_out[0] = sv[:, :hp]
    s_out[1] = sv[:, hp:]


def kernel(x, edge_index, W_self1, W_nei1, b1, W_self2, W_nei2, b2, Wc, bc):
    n, f = x.shape
    n_edges = edge_index.shape[1]
    ncls = Wc.shape[1]
    fp = 64
    f2 = f // 2
    fp2 = fp // 2
    e_w = n_edges // NS
    n_ch = e_w // CH
    rpt_buf = -(-n // NS)

    src = edge_index[0].astype(jnp.int32).reshape(NS, n_ch, CH)
    dst = edge_index[1].astype(jnp.int32).reshape(NS, n_ch, CH)
    zf = jnp.zeros((rpt_buf, f2), jnp.float32)
    z16 = jnp.zeros((rpt_buf, 16), jnp.float32)
    zfp = jnp.zeros((rpt_buf, fp2), jnp.float32)
    ones = jnp.ones((CH, 16), jnp.float32)
    wcp = jnp.concatenate([Wc, jnp.zeros((f, fp - ncls), jnp.float32)], axis=1)
    bcp = jnp.concatenate([bc, jnp.zeros((fp - ncls,), jnp.float32)]).reshape(1, fp)
    b1r = b1.reshape(1, f)
    b2r = b2.reshape(1, f)

    BLK = 1000
    grid = (n // BLK,)

    xh = jnp.stack([x[:, :f2], x[:, f2:]])
    aggx, deg = _make_sc_agg(n, n_edges, f2, True)(xh, src, dst, zf, ones, z16)

    p, s, rt = pl.pallas_call(
        _mid_body,
        grid=grid,
        in_specs=[
            pl.BlockSpec((BLK, f), lambda i: (i, 0)),
            pl.BlockSpec((2, BLK, f2), lambda i: (0, i, 0)),
            pl.BlockSpec((2, BLK, 16), lambda i: (0, i, 0)),
            pl.BlockSpec((f, f), lambda i: (0, 0)),
            pl.BlockSpec((1, f), lambda i: (0, 0)),
            pl.BlockSpec((f, f), lambda i: (0, 0)),
            pl.BlockSpec((f, f), lambda i: (0, 0)),
            pl.BlockSpec((f, f), lambda i: (0, 0)),
            pl.BlockSpec((f, fp), lambda i: (0, 0)),
            pl.BlockSpec((1, f), lambda i: (0, 0)),
            pl.BlockSpec((1, fp), lambda i: (0, 0)),
        ],
        out_specs=[pl.BlockSpec((2, BLK, fp2), lambda i: (0, i, 0)),
                   pl.BlockSpec((2, BLK, fp2), lambda i: (0, i, 0)),
                   pl.BlockSpec((BLK, 16), lambda i: (i, 0))],
        out_shape=[jax.ShapeDtypeStruct((2, n, fp2), jnp.float32),
                   jax.ShapeDtypeStruct((2, n, fp2), jnp.float32),
                   jax.ShapeDtypeStruct((n, 16), jnp.float32)],
    )(x, aggx, deg, W_self1, b1r, W_nei1, W_nei2, W_self2, wcp, b2r, bcp)

    (out2,) = _make_sc_agg(n, n_edges, fp2, False, final=True)(
        p, src, dst, zfp, s, rt)

    return jnp.concatenate([out2[0], out2[1]], axis=1)[:, :ncls]


# speedup vs baseline: 1.0503x; 1.0459x over previous
import functools

import jax
import jax.numpy as jnp
from jax import lax
from jax.experimental import pallas as pl
from jax.experimental.pallas import tpu as pltpu
from jax.experimental.pallas import tpu_sc as plsc

NC = 2
NS = 16
CH = 125


def _make_sc_agg(n_nodes, n_edges, feat2, with_deg, final=False,
                 packed_cols=False):
    e_w = n_edges // NS
    n_ch = e_w // CH
    rpt = (n_nodes // NS) & ~7
    rem = n_nodes - rpt * NS
    FB = rpt // 4

    out_type = [jax.ShapeDtypeStruct((NC, n_nodes, feat2), jnp.float32)]
    scratch = [
        pltpu.VMEM((n_ch, CH), jnp.int32),
        pltpu.VMEM((n_ch, CH), jnp.int32),
    ] + [pltpu.VMEM((CH, feat2), jnp.float32) for _ in range(4)] + [
        pltpu.VMEM_SHARED((n_nodes, feat2), jnp.float32),
    ] + [pltpu.SemaphoreType.DMA for _ in range(8)]
    if with_deg:
        out_type.append(jax.ShapeDtypeStruct((NC, n_nodes, 16), jnp.float32))
        scratch += [
            pltpu.VMEM((CH, 16), jnp.float32),
            pltpu.VMEM_SHARED((n_nodes, 16), jnp.float32),
        ]
    if final:
        scratch += [
            pltpu.VMEM((FB, feat2), jnp.float32),
            pltpu.VMEM((FB, 16), jnp.float32),
            pltpu.VMEM((FB, feat2), jnp.float32),
        ]

    mesh = plsc.VectorSubcoreMesh(core_axis_name="c", subcore_axis_name="s")

    half = n_ch // NC

    def body(g_hbm, src_hbm, dst_hbm, zf_hbm, *rest):
        sk_hbm = rt_hbm = sbuf = rbuf = abuf = None
        if with_deg:
            (ones_hbm, z16_hbm, out_hbm, deg_hbm, si, di,
             r0, r1, r2, r3, acc, g0, g1, g2, g3, s0, s1, s2, s3,
             ones_v, dacc) = rest
        elif final:
            (sk_hbm, rt_hbm, out_hbm, si, di, r0, r1, r2, r3, acc,
             g0, g1, g2, g3, s0, s1, s2, s3, sbuf, rbuf, abuf) = rest
        else:
            (out_hbm, si, di, r0, r1, r2, r3, acc,
             g0, g1, g2, g3, s0, s1, s2, s3) = rest
        rows = (r0, r1, r2, r3)
        gsem = (g0, g1, g2, g3)
        ssem = (s0, s1, s2, s3)
        c = lax.axis_index("c")
        s = lax.axis_index("s")
        pltpu.sync_copy(src_hbm.at[s], si)
        pltpu.sync_copy(dst_hbm.at[s], di)
        pltpu.sync_copy(zf_hbm.at[pl.ds(0, rpt)], acc.at[pl.ds(s * rpt, rpt)])
        if with_deg:
            pltpu.sync_copy(ones_hbm, ones_v)
            pltpu.sync_copy(z16_hbm.at[pl.ds(0, rpt)],
                            dacc.at[pl.ds(s * rpt, rpt)])

        @pl.when(s == NS - 1)
        def _zero_tail():
            pltpu.sync_copy(zf_hbm.at[pl.ds(0, rem)],
                            acc.at[pl.ds(NS * rpt, rem)])
            if with_deg:
                pltpu.sync_copy(z16_hbm.at[pl.ds(0, rem)],
                                dacc.at[pl.ds(NS * rpt, rem)])

        plsc.subcore_barrier()

        if packed_cols:
            g_half = g_hbm.at[:, pl.ds(c * feat2, feat2)]
        else:
            g_half = g_hbm.at[c]

        def wait_gather(j, b):
            pltpu.make_async_copy(
                g_half.at[si.at[j]], rows[b], gsem[b]).wait()

        def wait_scatter(j, b):
            pltpu.make_async_copy(rows[b], acc.at[di.at[j]], ssem[b]).wait()

        for b in range(2):
            pltpu.async_copy(g_half.at[si.at[b]], rows[b], gsem[b])

        def step(t, carry):
            for b4 in range(4):
                j = 4 * t + b4
                b = b4 % 4
                wait_gather(j, b)
                pltpu.async_copy(rows[b], acc.at[di.at[j]], ssem[b], add=True)
                if with_deg:
                    @pl.when((j >= c * half) & (j < (c + 1) * half))
                    def _deg_add():
                        pltpu.sync_copy(ones_v, dacc.at[di.at[j]], add=True)

                nb = (b + 2) % 4

                @pl.when(j + 2 < n_ch)
                def _next_gather():
                    @pl.when(j >= 2)
                    def _drain_prev():
                        wait_scatter(j - 2, nb)

                    pltpu.async_copy(
                        g_half.at[si.at[j + 2]], rows[nb], gsem[nb])
            return carry

        lax.fori_loop(0, n_ch // 4, step, 0)
        for k in range(4):
            j = n_ch - 4 + k
            wait_scatter(j, j % 4)
        plsc.subcore_barrier()

        if final:
            def combine(base, nrows):
                bs = pl.ds(0, nrows)
                pltpu.sync_copy(sk_hbm.at[c, pl.ds(base, nrows)], sbuf.at[bs])
                pltpu.sync_copy(rt_hbm.at[pl.ds(base, nrows)], rbuf.at[bs])
                pltpu.sync_copy(acc.at[pl.ds(base, nrows)], abuf.at[bs])

                def rowfn(r, carry):
                    rv = rbuf[r, pl.ds(0, 16)]
                    for g in range(feat2 // 16):
                        cs = pl.ds(g * 16, 16)
                        sbuf[r, cs] = sbuf[r, cs] + abuf[r, cs] * rv
                    return carry

                lax.fori_loop(0, nrows, rowfn, 0)
                pltpu.sync_copy(sbuf.at[bs], out_hbm.at[c, pl.ds(base, nrows)])

            for blk in range(4):
                combine(s * rpt + blk * FB, FB)

            @pl.when(s == NS - 1)
            def _fin_tail():
                combine(NS * rpt, rem)
        else:
            sl = pl.ds(s * rpt, rpt)
            pltpu.sync_copy(acc.at[sl], out_hbm.at[c, sl])
            if with_deg:
                pltpu.sync_copy(dacc.at[sl], deg_hbm.at[c, sl])

            @pl.when(s == NS - 1)
            def _write_tail():
                tl = pl.ds(NS * rpt, rem)
                pltpu.sync_copy(acc.at[tl], out_hbm.at[c, tl])
                if with_deg:
                    pltpu.sync_copy(dacc.at[tl], deg_hbm.at[c, tl])

    return pl.kernel(
        body, out_type=out_type, mesh=mesh, scratch_types=scratch,
        compiler_params=pltpu.CompilerParams(use_tc_tiling_on_sc=False))


def _mid_body(x_ref, ax_ref, deg_ref, ws1, b1r, wn1, wn2, ws2, wcp, b2r, bcr,
              p_out, s_out, rt_out):
    deg16 = jnp.maximum(deg_ref[0] + deg_ref[1], 1.0)
    rt_out[...] = 1.0 / deg16
    r = rt_out[:, :1]
    aggx = jnp.concatenate([ax_ref[0], ax_ref[1]], axis=1) * r
    dot = functools.partial(jnp.dot, preferred_element_type=jnp.float32)
    h = x_ref[...] @ ws1[...] + dot(aggx, wn1[...]) + b1r[...]
    h = jnp.maximum(h, 0.0)
    p = dot(dot(h, wn2[...]), wcp[...])
    sv = dot(dot(h, ws2[...]), wcp[...]) + dot(b2r[...], wcp[...]) + bcr[...]
    hp = p.shape[1] // 2
    p_out[0] = p[:, :hp]
    p_out[1] = p[:, hp:]
    s_out[...] = sv


def _fin_body(s_ref, p2_ref, rt_ref, o_ref):
    r = rt_ref[:, :1]
    o_ref[...] = s_ref[...] + jnp.concatenate(
        [p2_ref[0], p2_ref[1]], axis=1) * r


def kernel(x, edge_index, W_self1, W_nei1, b1, W_self2, W_nei2, b2, Wc, bc):
    n, f = x.shape
    n_edges = edge_index.shape[1]
    ncls = Wc.shape[1]
    fp = 64
    f2 = f // 2
    fp2 = fp // 2
    e_w = n_edges // NS
    n_ch = e_w // CH
    rpt_buf = -(-n // NS)

    src = edge_index[0].astype(jnp.int32).reshape(NS, n_ch, CH)
    dst = edge_index[1].astype(jnp.int32).reshape(NS, n_ch, CH)
    zf = jnp.zeros((rpt_buf, f2), jnp.float32)
    z16 = jnp.zeros((rpt_buf, 16), jnp.float32)
    zfp = jnp.zeros((rpt_buf, fp2), jnp.float32)
    ones = jnp.ones((CH, 16), jnp.float32)
    wcp = jnp.concatenate([Wc, jnp.zeros((f, fp - ncls), jnp.float32)], axis=1)
    bcp = jnp.concatenate([bc, jnp.zeros((fp - ncls,), jnp.float32)]).reshape(1, fp)
    b1r = b1.reshape(1, f)
    b2r = b2.reshape(1, f)

    BLK = 1000
    grid = (n // BLK,)

    xh = jnp.stack([x[:, :f2], x[:, f2:]])
    aggx, deg = _make_sc_agg(n, n_edges, f2, True)(xh, src, dst, zf, ones, z16)

    p, s, rt = pl.pallas_call(
        _mid_body,
        grid=grid,
        in_specs=[
            pl.BlockSpec((BLK, f), lambda i: (i, 0)),
            pl.BlockSpec((2, BLK, f2), lambda i: (0, i, 0)),
            pl.BlockSpec((2, BLK, 16), lambda i: (0, i, 0)),
            pl.BlockSpec((f, f), lambda i: (0, 0)),
            pl.BlockSpec((1, f), lambda i: (0, 0)),
            pl.BlockSpec((f, f), lambda i: (0, 0)),
            pl.BlockSpec((f, f), lambda i: (0, 0)),
            pl.BlockSpec((f, f), lambda i: (0, 0)),
            pl.BlockSpec((f, fp), lambda i: (0, 0)),
            pl.BlockSpec((1, f), lambda i: (0, 0)),
            pl.BlockSpec((1, fp), lambda i: (0, 0)),
        ],
        out_specs=[pl.BlockSpec((2, BLK, fp2), lambda i: (0, i, 0)),
                   pl.BlockSpec((BLK, fp), lambda i: (i, 0)),
                   pl.BlockSpec((BLK, 16), lambda i: (i, 0))],
        out_shape=[jax.ShapeDtypeStruct((2, n, fp2), jnp.float32),
                   jax.ShapeDtypeStruct((n, fp), jnp.float32),
                   jax.ShapeDtypeStruct((n, 16), jnp.float32)],
    )(x, aggx, deg, W_self1, b1r, W_nei1, W_nei2, W_self2, wcp, b2r, bcp)

    (agg2,) = _make_sc_agg(n, n_edges, fp2, False)(p, src, dst, zfp)

    out = pl.pallas_call(
        _fin_body,
        grid=grid,
        in_specs=[
            pl.BlockSpec((BLK, fp), lambda i: (i, 0)),
            pl.BlockSpec((2, BLK, fp2), lambda i: (0, i, 0)),
            pl.BlockSpec((BLK, 16), lambda i: (i, 0)),
        ],
        out_specs=pl.BlockSpec((BLK, fp), lambda i: (i, 0)),
        out_shape=jax.ShapeDtypeStruct((n, fp), jnp.float32),
    )(s, agg2, rt)

    return out[:, :ncls]


# speedup vs baseline: 1.1237x; 1.0698x over previous
import functools

import jax
import jax.numpy as jnp
from jax import lax
from jax.experimental import pallas as pl
from jax.experimental.pallas import tpu as pltpu
from jax.experimental.pallas import tpu_sc as plsc

NC = 2
NS = 16
CH = 125


def _make_sc_agg(n_nodes, n_edges, feat2, with_deg, final=False,
                 packed_cols=False):
    e_w = n_edges // NS
    n_ch = e_w // CH
    rpt = (n_nodes // NS) & ~7
    rem = n_nodes - rpt * NS
    FB = rpt // 4

    out_type = [jax.ShapeDtypeStruct((NC, n_nodes, feat2), jnp.float32)]
    scratch = [
        pltpu.VMEM((n_ch, CH), jnp.int32),
        pltpu.VMEM((n_ch, CH), jnp.int32),
    ] + [pltpu.VMEM((CH, feat2), jnp.float32) for _ in range(4)] + [
        pltpu.VMEM_SHARED((n_nodes, feat2), jnp.float32),
    ] + [pltpu.SemaphoreType.DMA for _ in range(8)]
    if with_deg:
        out_type.append(jax.ShapeDtypeStruct((NC, n_nodes, 16), jnp.float32))
        scratch += [
            pltpu.VMEM((CH, 16), jnp.float32),
            pltpu.VMEM_SHARED((n_nodes, 16), jnp.float32),
        ]
    if final:
        scratch += [
            pltpu.VMEM((FB, feat2), jnp.float32),
            pltpu.VMEM((FB, 16), jnp.float32),
            pltpu.VMEM((FB, feat2), jnp.float32),
        ]

    mesh = plsc.VectorSubcoreMesh(core_axis_name="c", subcore_axis_name="s")

    half = n_ch // NC

    def body(g_hbm, src_hbm, dst_hbm, zf_hbm, *rest):
        sk_hbm = rt_hbm = sbuf = rbuf = abuf = None
        if with_deg:
            (ones_hbm, z16_hbm, out_hbm, deg_hbm, si, di,
             r0, r1, r2, r3, acc, g0, g1, g2, g3, s0, s1, s2, s3,
             ones_v, dacc) = rest
        elif final:
            (sk_hbm, rt_hbm, out_hbm, si, di, r0, r1, r2, r3, acc,
             g0, g1, g2, g3, s0, s1, s2, s3, sbuf, rbuf, abuf) = rest
        else:
            (out_hbm, si, di, r0, r1, r2, r3, acc,
             g0, g1, g2, g3, s0, s1, s2, s3) = rest
        rows = (r0, r1, r2, r3)
        gsem = (g0, g1, g2, g3)
        ssem = (s0, s1, s2, s3)
        c = lax.axis_index("c")
        s = lax.axis_index("s")
        pltpu.sync_copy(src_hbm.at[s], si)
        pltpu.sync_copy(dst_hbm.at[s], di)
        pltpu.sync_copy(zf_hbm.at[pl.ds(0, rpt)], acc.at[pl.ds(s * rpt, rpt)])
        if with_deg:
            pltpu.sync_copy(ones_hbm, ones_v)
            pltpu.sync_copy(z16_hbm.at[pl.ds(0, rpt)],
                            dacc.at[pl.ds(s * rpt, rpt)])

        @pl.when(s == NS - 1)
        def _zero_tail():
            pltpu.sync_copy(zf_hbm.at[pl.ds(0, rem)],
                            acc.at[pl.ds(NS * rpt, rem)])
            if with_deg:
                pltpu.sync_copy(z16_hbm.at[pl.ds(0, rem)],
                                dacc.at[pl.ds(NS * rpt, rem)])

        plsc.subcore_barrier()

        if packed_cols:
            g_half = g_hbm.at[:, pl.ds(c * feat2, feat2)]
        else:
            g_half = g_hbm.at[c]

        def wait_gather(j, b):
            pltpu.make_async_copy(
                g_half.at[si.at[j]], rows[b], gsem[b]).wait()

        def wait_scatter(j, b):
            pltpu.make_async_copy(rows[b], acc.at[di.at[j]], ssem[b]).wait()

        for b in range(2):
            pltpu.async_copy(g_half.at[si.at[b]], rows[b], gsem[b])

        def step(t, carry):
            for b4 in range(4):
                j = 4 * t + b4
                b = b4 % 4
                wait_gather(j, b)
                pltpu.async_copy(rows[b], acc.at[di.at[j]], ssem[b], add=True)
                if with_deg:
                    @pl.when((j >= c * half) & (j < (c + 1) * half))
                    def _deg_add():
                        pltpu.sync_copy(ones_v, dacc.at[di.at[j]], add=True)

                nb = (b + 2) % 4

                @pl.when(j + 2 < n_ch)
                def _next_gather():
                    @pl.when(j >= 2)
                    def _drain_prev():
                        wait_scatter(j - 2, nb)

                    pltpu.async_copy(
                        g_half.at[si.at[j + 2]], rows[nb], gsem[nb])
            return carry

        lax.fori_loop(0, n_ch // 4, step, 0)
        for k in range(4):
            j = n_ch - 4 + k
            wait_scatter(j, j % 4)
        plsc.subcore_barrier()

        if final:
            def combine(base, nrows):
                bs = pl.ds(0, nrows)
                pltpu.sync_copy(sk_hbm.at[c, pl.ds(base, nrows)], sbuf.at[bs])
                pltpu.sync_copy(rt_hbm.at[pl.ds(base, nrows)], rbuf.at[bs])
                pltpu.sync_copy(acc.at[pl.ds(base, nrows)], abuf.at[bs])

                def rowfn(r, carry):
                    rv = rbuf[r, pl.ds(0, 16)]
                    for g in range(feat2 // 16):
                        cs = pl.ds(g * 16, 16)
                        sbuf[r, cs] = sbuf[r, cs] + abuf[r, cs] * rv
                    return carry

                lax.fori_loop(0, nrows, rowfn, 0)
                pltpu.sync_copy(sbuf.at[bs], out_hbm.at[c, pl.ds(base, nrows)])

            for blk in range(4):
                combine(s * rpt + blk * FB, FB)

            @pl.when(s == NS - 1)
            def _fin_tail():
                combine(NS * rpt, rem)
        else:
            sl = pl.ds(s * rpt, rpt)
            pltpu.sync_copy(acc.at[sl], out_hbm.at[c, sl])
            if with_deg:
                pltpu.sync_copy(dacc.at[sl], deg_hbm.at[c, sl])

            @pl.when(s == NS - 1)
            def _write_tail():
                tl = pl.ds(NS * rpt, rem)
                pltpu.sync_copy(acc.at[tl], out_hbm.at[c, tl])
                if with_deg:
                    pltpu.sync_copy(dacc.at[tl], deg_hbm.at[c, tl])

    return pl.kernel(
        body, out_type=out_type, mesh=mesh, scratch_types=scratch,
        compiler_params=pltpu.CompilerParams(use_tc_tiling_on_sc=False))


def _make_sc_agg_es(n_nodes, n_edges, feat):
    e_w = n_edges // (NC * NS)
    n_ch = e_w // CH
    rpt = (n_nodes // NS) & ~7
    rem = n_nodes - rpt * NS

    out_type = [jax.ShapeDtypeStruct((NC, n_nodes, feat), jnp.float32)]
    scratch = [
        pltpu.VMEM((n_ch, CH), jnp.int32),
        pltpu.VMEM((n_ch, CH), jnp.int32),
    ] + [pltpu.VMEM((CH, feat), jnp.float32) for _ in range(4)] + [
        pltpu.VMEM_SHARED((n_nodes, feat), jnp.float32),
    ] + [pltpu.SemaphoreType.DMA for _ in range(8)]

    mesh = plsc.VectorSubcoreMesh(core_axis_name="c", subcore_axis_name="s")

    def body(g_hbm, src_hbm, dst_hbm, zf_hbm, out_hbm, si, di,
             r0, r1, r2, r3, acc, g0, g1, g2, g3, s0, s1, s2, s3):
        rows = (r0, r1, r2, r3)
        gsem = (g0, g1, g2, g3)
        ssem = (s0, s1, s2, s3)
        c = lax.axis_index("c")
        s = lax.axis_index("s")
        pltpu.sync_copy(src_hbm.at[c, s], si)
        pltpu.sync_copy(dst_hbm.at[c, s], di)
        pltpu.sync_copy(zf_hbm.at[pl.ds(0, rpt)], acc.at[pl.ds(s * rpt, rpt)])

        @pl.when(s == NS - 1)
        def _zero_tail():
            pltpu.sync_copy(zf_hbm.at[pl.ds(0, rem)],
                            acc.at[pl.ds(NS * rpt, rem)])

        plsc.subcore_barrier()

        def wait_gather(j, b):
            pltpu.make_async_copy(
                g_hbm.at[si.at[j]], rows[b], gsem[b]).wait()

        def wait_scatter(j, b):
            pltpu.make_async_copy(rows[b], acc.at[di.at[j]], ssem[b]).wait()

        for b in range(2):
            pltpu.async_copy(g_hbm.at[si.at[b]], rows[b], gsem[b])

        def step(t, carry):
            for b in range(4):
                j = 4 * t + b
                wait_gather(j, b)
                pltpu.async_copy(rows[b], acc.at[di.at[j]], ssem[b], add=True)
                nb = (b + 2) % 4

                @pl.when(j + 2 < n_ch)
                def _next_gather():
                    @pl.when(j >= 2)
                    def _drain_prev():
                        wait_scatter(j - 2, nb)

                    pltpu.async_copy(g_hbm.at[si.at[j + 2]], rows[nb], gsem[nb])
            return carry

        lax.fori_loop(0, n_ch // 4, step, 0)
        for k in range(4):
            j = n_ch - 4 + k
            wait_scatter(j, j % 4)
        plsc.subcore_barrier()
        sl = pl.ds(s * rpt, rpt)
        pltpu.sync_copy(acc.at[sl], out_hbm.at[c, sl])

        @pl.when(s == NS - 1)
        def _write_tail():
            tl = pl.ds(NS * rpt, rem)
            pltpu.sync_copy(acc.at[tl], out_hbm.at[c, tl])

    return pl.kernel(
        body, out_type=out_type, mesh=mesh, scratch_types=scratch,
        compiler_params=pltpu.CompilerParams(use_tc_tiling_on_sc=False))


def _mid_body(x_ref, ax_ref, deg_ref, ws1, b1r, wn1, wn2, ws2, wcp, b2r, bcr,
              p_out, s_out, rt_out):
    deg16 = jnp.maximum(deg_ref[0] + deg_ref[1], 1.0)
    rt_out[...] = 1.0 / deg16
    r = rt_out[:, :1]
    aggx = jnp.concatenate([ax_ref[0], ax_ref[1]], axis=1) * r
    dot = functools.partial(jnp.dot, preferred_element_type=jnp.float32)
    h = x_ref[...] @ ws1[...] + dot(aggx, wn1[...]) + b1r[...]
    h = jnp.maximum(h, 0.0)
    p = dot(dot(h, wn2[...]), wcp[...])
    sv = dot(dot(h, ws2[...]), wcp[...]) + dot(b2r[...], wcp[...]) + bcr[...]
    p_out[...] = p
    s_out[...] = sv


def _fin_body(s_ref, p2_ref, rt_ref, o_ref):
    r = rt_ref[:, :1]
    o_ref[...] = s_ref[...] + (p2_ref[0] + p2_ref[1]) * r


def kernel(x, edge_index, W_self1, W_nei1, b1, W_self2, W_nei2, b2, Wc, bc):
    n, f = x.shape
    n_edges = edge_index.shape[1]
    ncls = Wc.shape[1]
    fp = 64
    f2 = f // 2
    fp2 = fp // 2
    e_w = n_edges // NS
    n_ch = e_w // CH
    rpt_buf = -(-n // NS)

    src = edge_index[0].astype(jnp.int32).reshape(NS, n_ch, CH)
    dst = edge_index[1].astype(jnp.int32).reshape(NS, n_ch, CH)
    src_es = src.reshape(NC, NS, n_ch // NC, CH)
    dst_es = dst.reshape(NC, NS, n_ch // NC, CH)
    zf = jnp.zeros((rpt_buf, f2), jnp.float32)
    z16 = jnp.zeros((rpt_buf, 16), jnp.float32)
    zfp = jnp.zeros((rpt_buf, fp), jnp.float32)
    ones = jnp.ones((CH, 16), jnp.float32)
    wcp = jnp.concatenate([Wc, jnp.zeros((f, fp - ncls), jnp.float32)], axis=1)
    bcp = jnp.concatenate([bc, jnp.zeros((fp - ncls,), jnp.float32)]).reshape(1, fp)
    b1r = b1.reshape(1, f)
    b2r = b2.reshape(1, f)

    BLK = 1000
    grid = (n // BLK,)

    xh = jnp.stack([x[:, :f2], x[:, f2:]])
    aggx, deg = _make_sc_agg(n, n_edges, f2, True)(xh, src, dst, zf, ones, z16)

    p, s, rt = pl.pallas_call(
        _mid_body,
        grid=grid,
        in_specs=[
            pl.BlockSpec((BLK, f), lambda i: (i, 0)),
            pl.BlockSpec((2, BLK, f2), lambda i: (0, i, 0)),
            pl.BlockSpec((2, BLK, 16), lambda i: (0, i, 0)),
            pl.BlockSpec((f, f), lambda i: (0, 0)),
            pl.BlockSpec((1, f), lambda i: (0, 0)),
            pl.BlockSpec((f, f), lambda i: (0, 0)),
            pl.BlockSpec((f, f), lambda i: (0, 0)),
            pl.BlockSpec((f, f), lambda i: (0, 0)),
            pl.BlockSpec((f, fp), lambda i: (0, 0)),
            pl.BlockSpec((1, f), lambda i: (0, 0)),
            pl.BlockSpec((1, fp), lambda i: (0, 0)),
        ],
        out_specs=[pl.BlockSpec((BLK, fp), lambda i: (i, 0)),
                   pl.BlockSpec((BLK, fp), lambda i: (i, 0)),
                   pl.BlockSpec((BLK, 16), lambda i: (i, 0))],
        out_shape=[jax.ShapeDtypeStruct((n, fp), jnp.float32),
                   jax.ShapeDtypeStruct((n, fp), jnp.float32),
                   jax.ShapeDtypeStruct((n, 16), jnp.float32)],
    )(x, aggx, deg, W_self1, b1r, W_nei1, W_nei2, W_self2, wcp, b2r, bcp)

    (agg2,) = _make_sc_agg_es(n, n_edges, fp)(p, src_es, dst_es, zfp)

    out = pl.pallas_call(
        _fin_body,
        grid=grid,
        in_specs=[
            pl.BlockSpec((BLK, fp), lambda i: (i, 0)),
            pl.BlockSpec((2, BLK, fp), lambda i: (0, i, 0)),
            pl.BlockSpec((BLK, 16), lambda i: (i, 0)),
        ],
        out_specs=pl.BlockSpec((BLK, fp), lambda i: (i, 0)),
        out_shape=jax.ShapeDtypeStruct((n, fp), jnp.float32),
    )(s, agg2, rt)

    return out[:, :ncls]


# speedup vs baseline: 1.1309x; 1.0064x over previous
import functools

import jax
import jax.numpy as jnp
from jax import lax
from jax.experimental import pallas as pl
from jax.experimental.pallas import tpu as pltpu
from jax.experimental.pallas import tpu_sc as plsc

NC = 2
NS = 16
CH = 125


def _make_sc_agg(n_nodes, n_edges, feat2, with_deg, final=False,
                 packed_cols=False):
    e_w = n_edges // NS
    n_ch = e_w // CH
    rpt = (n_nodes // NS) & ~7
    rem = n_nodes - rpt * NS
    FB = rpt // 4

    out_type = [jax.ShapeDtypeStruct((NC, n_nodes, feat2), jnp.float32)]
    scratch = [
        pltpu.VMEM((n_ch, CH), jnp.int32),
        pltpu.VMEM((n_ch, CH), jnp.int32),
    ] + [pltpu.VMEM((CH, feat2), jnp.float32) for _ in range(4)] + [
        pltpu.VMEM_SHARED((n_nodes, feat2), jnp.float32),
    ] + [pltpu.SemaphoreType.DMA for _ in range(8)]
    if with_deg:
        out_type.append(jax.ShapeDtypeStruct((NC, n_nodes, 16), jnp.float32))
        scratch += [
            pltpu.VMEM((CH, 16), jnp.float32),
            pltpu.VMEM_SHARED((n_nodes, 16), jnp.float32),
            pltpu.SemaphoreType.DMA,
            pltpu.SemaphoreType.DMA,
        ]
    if final:
        scratch += [
            pltpu.VMEM((FB, feat2), jnp.float32),
            pltpu.VMEM((FB, 16), jnp.float32),
            pltpu.VMEM((FB, feat2), jnp.float32),
        ]

    mesh = plsc.VectorSubcoreMesh(core_axis_name="c", subcore_axis_name="s")

    half = n_ch // NC

    def body(g_hbm, src_hbm, dst_hbm, zf_hbm, *rest):
        sk_hbm = rt_hbm = sbuf = rbuf = abuf = None
        if with_deg:
            (ones_hbm, z16_hbm, out_hbm, deg_hbm, si, di,
             r0, r1, r2, r3, acc, g0, g1, g2, g3, s0, s1, s2, s3,
             ones_v, dacc, d0, d1) = rest
            dsem = (d0, d1)
        elif final:
            (sk_hbm, rt_hbm, out_hbm, si, di, r0, r1, r2, r3, acc,
             g0, g1, g2, g3, s0, s1, s2, s3, sbuf, rbuf, abuf) = rest
        else:
            (out_hbm, si, di, r0, r1, r2, r3, acc,
             g0, g1, g2, g3, s0, s1, s2, s3) = rest
        rows = (r0, r1, r2, r3)
        gsem = (g0, g1, g2, g3)
        ssem = (s0, s1, s2, s3)
        c = lax.axis_index("c")
        s = lax.axis_index("s")
        pro = [
            pltpu.async_copy(src_hbm.at[s], si, gsem[0]),
            pltpu.async_copy(dst_hbm.at[s], di, gsem[1]),
            pltpu.async_copy(zf_hbm.at[pl.ds(0, rpt)],
                             acc.at[pl.ds(s * rpt, rpt)], gsem[2]),
        ]
        if with_deg:
            pro.append(pltpu.async_copy(ones_hbm, ones_v, gsem[3]))
            pro.append(pltpu.async_copy(z16_hbm.at[pl.ds(0, rpt)],
                                        dacc.at[pl.ds(s * rpt, rpt)], dsem[0]))
        for d in pro:
            d.wait()

        @pl.when(s == NS - 1)
        def _zero_tail():
            pltpu.sync_copy(zf_hbm.at[pl.ds(0, rem)],
                            acc.at[pl.ds(NS * rpt, rem)])
            if with_deg:
                pltpu.sync_copy(z16_hbm.at[pl.ds(0, rem)],
                                dacc.at[pl.ds(NS * rpt, rem)])

        plsc.subcore_barrier()

        if packed_cols:
            g_half = g_hbm.at[:, pl.ds(c * feat2, feat2)]
        else:
            g_half = g_hbm.at[c]

        def wait_gather(j, b):
            pltpu.make_async_copy(
                g_half.at[si.at[j]], rows[b], gsem[b]).wait()

        def wait_scatter(j, b):
            pltpu.make_async_copy(rows[b], acc.at[di.at[j]], ssem[b]).wait()

        for b in range(2):
            pltpu.async_copy(g_half.at[si.at[b]], rows[b], gsem[b])

        def step(t, carry):
            for b4 in range(4):
                j = 4 * t + b4
                b = b4 % 4
                wait_gather(j, b)
                pltpu.async_copy(rows[b], acc.at[di.at[j]], ssem[b], add=True)
                if with_deg:
                    @pl.when((j >= c * half) & (j < (c + 1) * half))
                    def _deg_add():
                        @pl.when(j >= c * half + 2)
                        def _deg_drain():
                            pltpu.make_async_copy(
                                ones_v, dacc.at[di.at[j - 2]],
                                dsem[b % 2]).wait()

                        pltpu.async_copy(ones_v, dacc.at[di.at[j]],
                                         dsem[b % 2], add=True)

                nb = (b + 2) % 4

                @pl.when(j + 2 < n_ch)
                def _next_gather():
                    @pl.when(j >= 2)
                    def _drain_prev():
                        wait_scatter(j - 2, nb)

                    pltpu.async_copy(
                        g_half.at[si.at[j + 2]], rows[nb], gsem[nb])
            return carry

        lax.fori_loop(0, n_ch // 4, step, 0)
        for k in range(4):
            j = n_ch - 4 + k
            wait_scatter(j, j % 4)
        if with_deg:
            for k in range(2):
                pltpu.make_async_copy(ones_v, dacc.at[di.at[k]],
                                      dsem[k]).wait()
        plsc.subcore_barrier()

        if final:
            def combine(base, nrows):
                bs = pl.ds(0, nrows)
                pltpu.sync_copy(sk_hbm.at[c, pl.ds(base, nrows)], sbuf.at[bs])
                pltpu.sync_copy(rt_hbm.at[pl.ds(base, nrows)], rbuf.at[bs])
                pltpu.sync_copy(acc.at[pl.ds(base, nrows)], abuf.at[bs])

                def rowfn(r, carry):
                    rv = rbuf[r, pl.ds(0, 16)]
                    for g in range(feat2 // 16):
                        cs = pl.ds(g * 16, 16)
                        sbuf[r, cs] = sbuf[r, cs] + abuf[r, cs] * rv
                    return carry

                lax.fori_loop(0, nrows, rowfn, 0)
                pltpu.sync_copy(sbuf.at[bs], out_hbm.at[c, pl.ds(base, nrows)])

            for blk in range(4):
                combine(s * rpt + blk * FB, FB)

            @pl.when(s == NS - 1)
            def _fin_tail():
                combine(NS * rpt, rem)
        else:
            sl = pl.ds(s * rpt, rpt)
            pltpu.sync_copy(acc.at[sl], out_hbm.at[c, sl])
            if with_deg:
                pltpu.sync_copy(dacc.at[sl], deg_hbm.at[c, sl])

            @pl.when(s == NS - 1)
            def _write_tail():
                tl = pl.ds(NS * rpt, rem)
                pltpu.sync_copy(acc.at[tl], out_hbm.at[c, tl])
                if with_deg:
                    pltpu.sync_copy(dacc.at[tl], deg_hbm.at[c, tl])

    return pl.kernel(
        body, out_type=out_type, mesh=mesh, scratch_types=scratch,
        compiler_params=pltpu.CompilerParams(use_tc_tiling_on_sc=False))


def _make_sc_agg_es(n_nodes, n_edges, feat):
    e_w = n_edges // (NC * NS)
    n_ch = e_w // CH
    rpt = (n_nodes // NS) & ~7
    rem = n_nodes - rpt * NS

    out_type = [jax.ShapeDtypeStruct((NC, n_nodes, feat), jnp.float32)]
    scratch = [
        pltpu.VMEM((n_ch, CH), jnp.int32),
        pltpu.VMEM((n_ch, CH), jnp.int32),
    ] + [pltpu.VMEM((CH, feat), jnp.float32) for _ in range(4)] + [
        pltpu.VMEM_SHARED((n_nodes, feat), jnp.float32),
    ] + [pltpu.SemaphoreType.DMA for _ in range(8)]

    mesh = plsc.VectorSubcoreMesh(core_axis_name="c", subcore_axis_name="s")

    def body(g_hbm, src_hbm, dst_hbm, zf_hbm, out_hbm, si, di,
             r0, r1, r2, r3, acc, g0, g1, g2, g3, s0, s1, s2, s3):
        rows = (r0, r1, r2, r3)
        gsem = (g0, g1, g2, g3)
        ssem = (s0, s1, s2, s3)
        c = lax.axis_index("c")
        s = lax.axis_index("s")
        pro = [
            pltpu.async_copy(src_hbm.at[c, s], si, gsem[0]),
            pltpu.async_copy(dst_hbm.at[c, s], di, gsem[1]),
            pltpu.async_copy(zf_hbm.at[pl.ds(0, rpt)],
                             acc.at[pl.ds(s * rpt, rpt)], gsem[2]),
        ]
        for d in pro:
            d.wait()

        @pl.when(s == NS - 1)
        def _zero_tail():
            pltpu.sync_copy(zf_hbm.at[pl.ds(0, rem)],
                            acc.at[pl.ds(NS * rpt, rem)])

        plsc.subcore_barrier()

        def wait_gather(j, b):
            pltpu.make_async_copy(
                g_hbm.at[si.at[j]], rows[b], gsem[b]).wait()

        def wait_scatter(j, b):
            pltpu.make_async_copy(rows[b], acc.at[di.at[j]], ssem[b]).wait()

        for b in range(2):
            pltpu.async_copy(g_hbm.at[si.at[b]], rows[b], gsem[b])

        def step(t, carry):
            for b in range(4):
                j = 4 * t + b
                wait_gather(j, b)
                pltpu.async_copy(rows[b], acc.at[di.at[j]], ssem[b], add=True)
                nb = (b + 2) % 4

                @pl.when(j + 2 < n_ch)
                def _next_gather():
                    @pl.when(j >= 2)
                    def _drain_prev():
                        wait_scatter(j - 2, nb)

                    pltpu.async_copy(g_hbm.at[si.at[j + 2]], rows[nb], gsem[nb])
            return carry

        lax.fori_loop(0, n_ch // 4, step, 0)
        for k in range(4):
            j = n_ch - 4 + k
            wait_scatter(j, j % 4)
        plsc.subcore_barrier()
        sl = pl.ds(s * rpt, rpt)
        pltpu.sync_copy(acc.at[sl], out_hbm.at[c, sl])

        @pl.when(s == NS - 1)
        def _write_tail():
            tl = pl.ds(NS * rpt, rem)
            pltpu.sync_copy(acc.at[tl], out_hbm.at[c, tl])

    return pl.kernel(
        body, out_type=out_type, mesh=mesh, scratch_types=scratch,
        compiler_params=pltpu.CompilerParams(use_tc_tiling_on_sc=False))


def _mid_body(x_ref, ax_ref, deg_ref, ws1, b1r, wn1, wn2, ws2, wcp, b2r, bcr,
              p_out, s_out, rt_out):
    deg16 = jnp.maximum(deg_ref[0] + deg_ref[1], 1.0)
    rt_out[...] = 1.0 / deg16
    r = rt_out[:, :1]
    aggx = jnp.concatenate([ax_ref[0], ax_ref[1]], axis=1) * r
    dot = functools.partial(jnp.dot, preferred_element_type=jnp.float32)
    h = x_ref[...] @ ws1[...] + dot(aggx, wn1[...]) + b1r[...]
    h = jnp.maximum(h, 0.0)
    p = dot(dot(h, wn2[...]), wcp[...])
    sv = dot(dot(h, ws2[...]), wcp[...]) + dot(b2r[...], wcp[...]) + bcr[...]
    p_out[...] = p
    s_out[...] = sv


def _fin_body(s_ref, p2_ref, rt_ref, o_ref):
    r = rt_ref[:, :1]
    o_ref[...] = s_ref[...] + (p2_ref[0] + p2_ref[1]) * r


def kernel(x, edge_index, W_self1, W_nei1, b1, W_self2, W_nei2, b2, Wc, bc):
    n, f = x.shape
    n_edges = edge_index.shape[1]
    ncls = Wc.shape[1]
    fp = 64
    f2 = f // 2
    fp2 = fp // 2
    e_w = n_edges // NS
    n_ch = e_w // CH
    rpt_buf = -(-n // NS)

    src = edge_index[0].astype(jnp.int32).reshape(NS, n_ch, CH)
    dst = edge_index[1].astype(jnp.int32).reshape(NS, n_ch, CH)
    src_es = src.reshape(NC, NS, n_ch // NC, CH)
    dst_es = dst.reshape(NC, NS, n_ch // NC, CH)
    zf = jnp.zeros((rpt_buf, f2), jnp.float32)
    z16 = jnp.zeros((rpt_buf, 16), jnp.float32)
    zfp = jnp.zeros((rpt_buf, fp), jnp.float32)
    ones = jnp.ones((CH, 16), jnp.float32)
    wcp = jnp.concatenate([Wc, jnp.zeros((f, fp - ncls), jnp.float32)], axis=1)
    bcp = jnp.concatenate([bc, jnp.zeros((fp - ncls,), jnp.float32)]).reshape(1, fp)
    b1r = b1.reshape(1, f)
    b2r = b2.reshape(1, f)

    BLK = 2000
    grid = (n // BLK,)

    xh = jnp.stack([x[:, :f2], x[:, f2:]])
    aggx, deg = _make_sc_agg(n, n_edges, f2, True)(xh, src, dst, zf, ones, z16)

    p, s, rt = pl.pallas_call(
        _mid_body,
        grid=grid,
        in_specs=[
            pl.BlockSpec((BLK, f), lambda i: (i, 0)),
            pl.BlockSpec((2, BLK, f2), lambda i: (0, i, 0)),
            pl.BlockSpec((2, BLK, 16), lambda i: (0, i, 0)),
            pl.BlockSpec((f, f), lambda i: (0, 0)),
            pl.BlockSpec((1, f), lambda i: (0, 0)),
            pl.BlockSpec((f, f), lambda i: (0, 0)),
            pl.BlockSpec((f, f), lambda i: (0, 0)),
            pl.BlockSpec((f, f), lambda i: (0, 0)),
            pl.BlockSpec((f, fp), lambda i: (0, 0)),
            pl.BlockSpec((1, f), lambda i: (0, 0)),
            pl.BlockSpec((1, fp), lambda i: (0, 0)),
        ],
        out_specs=[pl.BlockSpec((BLK, fp), lambda i: (i, 0)),
                   pl.BlockSpec((BLK, fp), lambda i: (i, 0)),
                   pl.BlockSpec((BLK, 16), lambda i: (i, 0))],
        out_shape=[jax.ShapeDtypeStruct((n, fp), jnp.float32),
                   jax.ShapeDtypeStruct((n, fp), jnp.float32),
                   jax.ShapeDtypeStruct((n, 16), jnp.float32)],
    )(x, aggx, deg, W_self1, b1r, W_nei1, W_nei2, W_self2, wcp, b2r, bcp)

    (agg2,) = _make_sc_agg_es(n, n_edges, fp)(p, src_es, dst_es, zfp)

    out = pl.pallas_call(
        _fin_body,
        grid=grid,
        in_specs=[
            pl.BlockSpec((BLK, fp), lambda i: (i, 0)),
            pl.BlockSpec((2, BLK, fp), lambda i: (0, i, 0)),
            pl.BlockSpec((BLK, 16), lambda i: (i, 0)),
        ],
        out_specs=pl.BlockSpec((BLK, fp), lambda i: (i, 0)),
        out_shape=jax.ShapeDtypeStruct((n, fp), jnp.float32),
    )(s, agg2, rt)

    return out[:, :ncls]


# speedup vs baseline: 1.1327x; 1.0016x over previous
import functools

import jax
import jax.numpy as jnp
from jax import lax
from jax.experimental import pallas as pl
from jax.experimental.pallas import tpu as pltpu
from jax.experimental.pallas import tpu_sc as plsc

NC = 2
NS = 16
CH = 125


def _make_sc_agg(n_nodes, n_edges, feat2, with_deg):
    e_w = n_edges // NS
    n_ch = e_w // CH
    rpt = (n_nodes // NS) & ~7
    rem = n_nodes - rpt * NS

    out_type = [jax.ShapeDtypeStruct((NC, n_nodes, feat2), jnp.float32)]
    scratch = [
        pltpu.VMEM((n_ch, CH), jnp.int32),
        pltpu.VMEM((n_ch, CH), jnp.int32),
    ] + [pltpu.VMEM((CH, feat2), jnp.float32) for _ in range(4)] + [
        pltpu.VMEM_SHARED((n_nodes, feat2), jnp.float32),
    ] + [pltpu.SemaphoreType.DMA for _ in range(8)]
    if with_deg:
        out_type.append(jax.ShapeDtypeStruct((NC, n_nodes, 16), jnp.float32))
        scratch += [
            pltpu.VMEM((CH, 16), jnp.float32),
            pltpu.VMEM_SHARED((n_nodes, 16), jnp.float32),
            pltpu.SemaphoreType.DMA,
            pltpu.SemaphoreType.DMA,
        ]

    mesh = plsc.VectorSubcoreMesh(core_axis_name="c", subcore_axis_name="s")

    half = n_ch // NC

    def body(g_hbm, src_hbm, dst_hbm, zf_hbm, *rest):
        if with_deg:
            (ones_hbm, z16_hbm, out_hbm, deg_hbm, si, di,
             r0, r1, r2, r3, acc, g0, g1, g2, g3, s0, s1, s2, s3,
             ones_v, dacc, d0, d1) = rest
            dsem = (d0, d1)
        else:
            (out_hbm, si, di, r0, r1, r2, r3, acc,
             g0, g1, g2, g3, s0, s1, s2, s3) = rest
        rows = (r0, r1, r2, r3)
        gsem = (g0, g1, g2, g3)
        ssem = (s0, s1, s2, s3)
        c = lax.axis_index("c")
        s = lax.axis_index("s")
        pro = [
            pltpu.async_copy(src_hbm.at[s], si, gsem[0]),
            pltpu.async_copy(dst_hbm.at[s], di, gsem[1]),
            pltpu.async_copy(zf_hbm.at[pl.ds(0, rpt)],
                             acc.at[pl.ds(s * rpt, rpt)], gsem[2]),
        ]
        if with_deg:
            pro.append(pltpu.async_copy(ones_hbm, ones_v, gsem[3]))
            pro.append(pltpu.async_copy(z16_hbm.at[pl.ds(0, rpt)],
                                        dacc.at[pl.ds(s * rpt, rpt)], dsem[0]))
        for d in pro:
            d.wait()

        @pl.when(s == NS - 1)
        def _zero_tail():
            pltpu.sync_copy(zf_hbm.at[pl.ds(0, rem)],
                            acc.at[pl.ds(NS * rpt, rem)])
            if with_deg:
                pltpu.sync_copy(z16_hbm.at[pl.ds(0, rem)],
                                dacc.at[pl.ds(NS * rpt, rem)])

        plsc.subcore_barrier()

        g_half = g_hbm.at[c]

        def wait_gather(j, b):
            pltpu.make_async_copy(
                g_half.at[si.at[j]], rows[b], gsem[b]).wait()

        def wait_scatter(j, b):
            pltpu.make_async_copy(rows[b], acc.at[di.at[j]], ssem[b]).wait()

        for b in range(2):
            pltpu.async_copy(g_half.at[si.at[b]], rows[b], gsem[b])

        def step(t, carry):
            for b4 in range(4):
                j = 4 * t + b4
                b = b4 % 4
                wait_gather(j, b)
                pltpu.async_copy(rows[b], acc.at[di.at[j]], ssem[b], add=True)
                if with_deg:
                    @pl.when((j >= c * half) & (j < (c + 1) * half))
                    def _deg_add():
                        @pl.when(j >= c * half + 2)
                        def _deg_drain():
                            pltpu.make_async_copy(
                                ones_v, dacc.at[di.at[j - 2]],
                                dsem[b % 2]).wait()

                        pltpu.async_copy(ones_v, dacc.at[di.at[j]],
                                         dsem[b % 2], add=True)

                nb = (b + 2) % 4

                @pl.when(j + 2 < n_ch)
                def _next_gather():
                    @pl.when(j >= 2)
                    def _drain_prev():
                        wait_scatter(j - 2, nb)

                    pltpu.async_copy(
                        g_half.at[si.at[j + 2]], rows[nb], gsem[nb])
            return carry

        lax.fori_loop(0, n_ch // 4, step, 0)
        for k in range(4):
            j = n_ch - 4 + k
            wait_scatter(j, j % 4)
        if with_deg:
            for k in range(2):
                pltpu.make_async_copy(ones_v, dacc.at[di.at[k]],
                                      dsem[k]).wait()
        plsc.subcore_barrier()

        sl = pl.ds(s * rpt, rpt)
        pltpu.sync_copy(acc.at[sl], out_hbm.at[c, sl])
        if with_deg:
            pltpu.sync_copy(dacc.at[sl], deg_hbm.at[c, sl])

        @pl.when(s == NS - 1)
        def _write_tail():
            tl = pl.ds(NS * rpt, rem)
            pltpu.sync_copy(acc.at[tl], out_hbm.at[c, tl])
            if with_deg:
                pltpu.sync_copy(dacc.at[tl], deg_hbm.at[c, tl])

    return pl.kernel(
        body, out_type=out_type, mesh=mesh, scratch_types=scratch,
        compiler_params=pltpu.CompilerParams(use_tc_tiling_on_sc=False))


def _make_sc_agg_es(n_nodes, n_edges, feat):
    e_w = n_edges // (NC * NS)
    n_ch = e_w // CH
    rpt = (n_nodes // NS) & ~7
    rem = n_nodes - rpt * NS

    out_type = [jax.ShapeDtypeStruct((NC, n_nodes, feat), jnp.float32)]
    scratch = [
        pltpu.VMEM((n_ch, CH), jnp.int32),
        pltpu.VMEM((n_ch, CH), jnp.int32),
    ] + [pltpu.VMEM((CH, feat), jnp.float32) for _ in range(4)] + [
        pltpu.VMEM_SHARED((n_nodes, feat), jnp.float32),
    ] + [pltpu.SemaphoreType.DMA for _ in range(8)]

    mesh = plsc.VectorSubcoreMesh(core_axis_name="c", subcore_axis_name="s")

    def body(g_hbm, src_hbm, dst_hbm, zf_hbm, out_hbm, si, di,
             r0, r1, r2, r3, acc, g0, g1, g2, g3, s0, s1, s2, s3):
        rows = (r0, r1, r2, r3)
        gsem = (g0, g1, g2, g3)
        ssem = (s0, s1, s2, s3)
        c = lax.axis_index("c")
        s = lax.axis_index("s")
        pro = [
            pltpu.async_copy(src_hbm.at[c, s], si, gsem[0]),
            pltpu.async_copy(dst_hbm.at[c, s], di, gsem[1]),
            pltpu.async_copy(zf_hbm.at[pl.ds(0, rpt)],
                             acc.at[pl.ds(s * rpt, rpt)], gsem[2]),
        ]
        for d in pro:
            d.wait()

        @pl.when(s == NS - 1)
        def _zero_tail():
            pltpu.sync_copy(zf_hbm.at[pl.ds(0, rem)],
                            acc.at[pl.ds(NS * rpt, rem)])

        plsc.subcore_barrier()

        def wait_gather(j, b):
            pltpu.make_async_copy(
                g_hbm.at[si.at[j]], rows[b], gsem[b]).wait()

        def wait_scatter(j, b):
            pltpu.make_async_copy(rows[b], acc.at[di.at[j]], ssem[b]).wait()

        for b in range(2):
            pltpu.async_copy(g_hbm.at[si.at[b]], rows[b], gsem[b])

        def step(t, carry):
            for b in range(4):
                j = 4 * t + b
                wait_gather(j, b)
                pltpu.async_copy(rows[b], acc.at[di.at[j]], ssem[b], add=True)
                nb = (b + 2) % 4

                @pl.when(j + 2 < n_ch)
                def _next_gather():
                    @pl.when(j >= 2)
                    def _drain_prev():
                        wait_scatter(j - 2, nb)

                    pltpu.async_copy(g_hbm.at[si.at[j + 2]], rows[nb], gsem[nb])
            return carry

        lax.fori_loop(0, n_ch // 4, step, 0)
        for k in range(4):
            j = n_ch - 4 + k
            wait_scatter(j, j % 4)
        plsc.subcore_barrier()
        sl = pl.ds(s * rpt, rpt)
        pltpu.sync_copy(acc.at[sl], out_hbm.at[c, sl])

        @pl.when(s == NS - 1)
        def _write_tail():
            tl = pl.ds(NS * rpt, rem)
            pltpu.sync_copy(acc.at[tl], out_hbm.at[c, tl])

    return pl.kernel(
        body, out_type=out_type, mesh=mesh, scratch_types=scratch,
        compiler_params=pltpu.CompilerParams(use_tc_tiling_on_sc=False))


def _mid_body(x_ref, ax_ref, deg_ref, ws1, b1r, wn1, wn2, ws2, wcp, b2r, bcr,
              p_out, s_out, rt_out):
    deg16 = jnp.maximum(deg_ref[0] + deg_ref[1], 1.0)
    rt_out[...] = 1.0 / deg16
    r = rt_out[:, :1]
    aggx = jnp.concatenate([ax_ref[0], ax_ref[1]], axis=1) * r
    dot = functools.partial(jnp.dot, preferred_element_type=jnp.float32)
    h = x_ref[...] @ ws1[...] + dot(aggx, wn1[...]) + b1r[...]
    h = jnp.maximum(h, 0.0)
    p = dot(dot(h, wn2[...]), wcp[...])
    sv = dot(dot(h, ws2[...]), wcp[...]) + dot(b2r[...], wcp[...]) + bcr[...]
    p_out[...] = p
    s_out[...] = sv


def _fin_body(s_ref, p2_ref, rt_ref, o_ref):
    r = rt_ref[:, :1]
    o_ref[...] = s_ref[...] + (p2_ref[0] + p2_ref[1]) * r


def kernel(x, edge_index, W_self1, W_nei1, b1, W_self2, W_nei2, b2, Wc, bc):
    n, f = x.shape
    n_edges = edge_index.shape[1]
    ncls = Wc.shape[1]
    fp = 64
    f2 = f // 2
    e_w = n_edges // NS
    n_ch = e_w // CH
    rpt_buf = -(-n // NS)

    src = edge_index[0].astype(jnp.int32).reshape(NS, n_ch, CH)
    dst = edge_index[1].astype(jnp.int32).reshape(NS, n_ch, CH)
    src_es = src.reshape(NC, NS, n_ch // NC, CH)
    dst_es = dst.reshape(NC, NS, n_ch // NC, CH)
    zf = jnp.zeros((rpt_buf, f2), jnp.float32)
    z16 = jnp.zeros((rpt_buf, 16), jnp.float32)
    zfp = jnp.zeros((rpt_buf, fp), jnp.float32)
    ones = jnp.ones((CH, 16), jnp.float32)
    wcp = jnp.concatenate([Wc, jnp.zeros((f, fp - ncls), jnp.float32)], axis=1)
    bcp = jnp.concatenate([bc, jnp.zeros((fp - ncls,), jnp.float32)]).reshape(1, fp)
    b1r = b1.reshape(1, f)
    b2r = b2.reshape(1, f)

    BLK = 2000
    grid = (n // BLK,)

    xh = jnp.stack([x[:, :f2], x[:, f2:]])
    aggx, deg = _make_sc_agg(n, n_edges, f2, True)(xh, src, dst, zf, ones, z16)

    p, s, rt = pl.pallas_call(
        _mid_body,
        grid=grid,
        in_specs=[
            pl.BlockSpec((BLK, f), lambda i: (i, 0)),
            pl.BlockSpec((2, BLK, f2), lambda i: (0, i, 0)),
            pl.BlockSpec((2, BLK, 16), lambda i: (0, i, 0)),
            pl.BlockSpec((f, f), lambda i: (0, 0)),
            pl.BlockSpec((1, f), lambda i: (0, 0)),
            pl.BlockSpec((f, f), lambda i: (0, 0)),
            pl.BlockSpec((f, f), lambda i: (0, 0)),
            pl.BlockSpec((f, f), lambda i: (0, 0)),
            pl.BlockSpec((f, fp), lambda i: (0, 0)),
            pl.BlockSpec((1, f), lambda i: (0, 0)),
            pl.BlockSpec((1, fp), lambda i: (0, 0)),
        ],
        out_specs=[pl.BlockSpec((BLK, fp), lambda i: (i, 0)),
                   pl.BlockSpec((BLK, fp), lambda i: (i, 0)),
                   pl.BlockSpec((BLK, 16), lambda i: (i, 0))],
        out_shape=[jax.ShapeDtypeStruct((n, fp), jnp.float32),
                   jax.ShapeDtypeStruct((n, fp), jnp.float32),
                   jax.ShapeDtypeStruct((n, 16), jnp.float32)],
    )(x, aggx, deg, W_self1, b1r, W_nei1, W_nei2, W_self2, wcp, b2r, bcp)

    (agg2,) = _make_sc_agg_es(n, n_edges, fp)(p, src_es, dst_es, zfp)

    out = pl.pallas_call(
        _fin_body,
        grid=grid,
        in_specs=[
            pl.BlockSpec((BLK, fp), lambda i: (i, 0)),
            pl.BlockSpec((2, BLK, fp), lambda i: (0, i, 0)),
            pl.BlockSpec((BLK, 16), lambda i: (i, 0)),
        ],
        out_specs=pl.BlockSpec((BLK, fp), lambda i: (i, 0)),
        out_shape=jax.ShapeDtypeStruct((n, fp), jnp.float32),
    )(s, agg2, rt)

    return out[:, :ncls]
